# baseline (speedup 1.0000x reference)
import functools

import jax
import jax.numpy as jnp
from jax import lax
from jax.experimental import pallas as pl
from jax.experimental.pallas import tpu as pltpu
from jax.experimental.pallas import tpu_sc as plsc
import numpy as np

BATCH = 64
OBS = 128 * 1024
ROW_F = OBS * 3
NW = 32
ROWS_PER_W = BATCH // NW

CHUNK = 49152
GROUPS = CHUNK // 48
UNROLL = 8
CHUNKS_PER_ROW = ROW_F // CHUNK

THRESH = float(2.0 * np.sin(np.deg2rad(2.0) / 2.0))
THRESH2 = THRESH ** 2
ALPHA = 1.0
BETA = 1.0


def _sc_partials_kernel(z_hbm, r_hbm, part_hbm, r_v, buf_v, acc_v):
    wid = lax.axis_index("s") * 2 + lax.axis_index("c")
    pltpu.sync_copy(r_hbm, r_v)

    iota = lax.broadcasted_iota(jnp.int32, (16,), 0)
    idx0 = iota * 3

    for rr in range(ROWS_PER_W):
        b = wid * ROWS_PER_W + rr
        rvec = plsc.load_gather(r_v, [jnp.full((16,), b, jnp.int32)])
        bvec = -0.5 / (rvec * rvec)
        row_base = b * ROW_F

        acc = jnp.zeros((16,), jnp.float32)
        for c in range(CHUNKS_PER_ROW):
            pltpu.sync_copy(z_hbm.at[pl.ds(row_base + c * CHUNK, CHUNK)], buf_v)

            def step(i, acc):
                base = i * (48 * UNROLL)
                for k in range(UNROLL):
                    off = base + k * 48
                    gx = plsc.load_gather(buf_v, [idx0 + off])
                    gy = plsc.load_gather(buf_v, [idx0 + (off + 1)])
                    gz = plsc.load_gather(buf_v, [idx0 + (off + 2)])
                    z2 = gx * gx + gy * gy + gz * gz
                    e = jnp.exp(bvec * z2)
                    acc = acc + jnp.where(z2 < THRESH2, e, 0.0)
                return acc

            acc = lax.fori_loop(0, GROUPS // UNROLL, step, acc)

        acc_v[...] = acc
        pltpu.sync_copy(acc_v, part_hbm.at[b])


@functools.partial(jax.jit, static_argnames=())
def _sc_partials(zflat, r):
    kfn = pl.kernel(
        _sc_partials_kernel,
        out_type=jax.ShapeDtypeStruct((BATCH, 16), jnp.float32),
        mesh=plsc.VectorSubcoreMesh(core_axis_name="c", subcore_axis_name="s"),
        scratch_types=[
            pltpu.VMEM((BATCH,), jnp.float32),
            pltpu.VMEM((CHUNK,), jnp.float32),
            pltpu.VMEM((16,), jnp.float32),
        ],
        compiler_params=pltpu.CompilerParams(needs_layout_passes=False),
    )
    return kfn(zflat, r)


def _tc_epilogue_kernel(part_ref, r_ref, n_ref, raw_ref, mu_ref, s2_ref, obj_ref):
    raw = jnp.sum(part_ref[...], axis=1, keepdims=True)
    r = r_ref[...]
    n = n_ref[0, 0]
    a = 1.0 / (r * r)
    lam = (0.5 * THRESH2) * a
    mu_per = (1.0 - jnp.exp(-lam)) / lam
    e2 = (1.0 - jnp.exp(-2.0 * lam)) / (2.0 * lam)
    sig2_per = e2 - mu_per * mu_per
    mu = n * mu_per
    sigma2 = n * sig2_per
    sigma = jnp.sqrt(sigma2)
    raw_ref[...] = raw
    mu_ref[...] = mu
    s2_ref[...] = sigma2
    obj_ref[...] = raw - ALPHA * mu - BETA + sigma


def _tc_epilogue(part, r, num_obs):
    out_shape = jax.ShapeDtypeStruct((BATCH, 1), jnp.float32)
    return pl.pallas_call(
        _tc_epilogue_kernel,
        out_shape=(out_shape, out_shape, out_shape, out_shape),
        in_specs=[
            pl.BlockSpec(memory_space=pltpu.VMEM),
            pl.BlockSpec(memory_space=pltpu.VMEM),
            pl.BlockSpec(memory_space=pltpu.SMEM),
        ],
        out_specs=(
            pl.BlockSpec(memory_space=pltpu.VMEM),
            pl.BlockSpec(memory_space=pltpu.VMEM),
            pl.BlockSpec(memory_space=pltpu.VMEM),
            pl.BlockSpec(memory_space=pltpu.VMEM),
        ),
    )(part, r, num_obs)


def kernel(z, R, num_obs):
    zflat = jnp.reshape(z, (-1,))
    part = _sc_partials(zflat, R)
    n2 = jnp.reshape(jnp.asarray(num_obs, jnp.float32), (1, 1))
    raw, mu, sigma2, obj = _tc_epilogue(part, jnp.reshape(R, (BATCH, 1)), n2)
    return (jnp.reshape(raw, (BATCH,)), jnp.reshape(mu, (BATCH,)),
            jnp.reshape(sigma2, (BATCH,)), jnp.reshape(obj, (BATCH,)))


# speedup vs baseline: 218.3055x; 218.3055x over previous
import functools

import jax
import jax.numpy as jnp
from jax import lax
from jax.experimental import pallas as pl
from jax.experimental.pallas import tpu as pltpu
from jax.experimental.pallas import tpu_sc as plsc
import numpy as np

BATCH = 64
NW = 32
ROWS_PER_W = BATCH // NW

CROWS = 16
NCHUNK = 128 // CROWS
GUNROLL = 8

THRESH = float(2.0 * np.sin(np.deg2rad(2.0) / 2.0))
THRESH2 = THRESH ** 2
ALPHA = 1.0
BETA = 1.0


def _sc_partials_kernel(z_hbm, r_hbm, part_hbm, r_v, bufs, acc_v, sems):
    wid = lax.axis_index("s") * 2 + lax.axis_index("c")
    pltpu.sync_copy(r_hbm, r_v)

    def start(b, c, par):
        return [
            pltpu.async_copy(
                z_hbm.at[b, comp, pl.ds(c * CROWS, CROWS), :],
                bufs[par][comp], sems[par])
            for comp in range(3)
        ]

    for rr in range(ROWS_PER_W):
        b = wid * ROWS_PER_W + rr
        rvec = plsc.load_gather(r_v, [jnp.full((16,), b, jnp.int32)])
        bvec = -0.5 / (rvec * rvec)

        acc = jnp.zeros((16,), jnp.float32)
        descs = start(b, 0, 0)
        for c in range(NCHUNK):
            par = c % 2
            if c + 1 < NCHUNK:
                nxt = start(b, c + 1, 1 - par)
            for d in descs:
                d.wait()
            bx, by, bz = bufs[par]

            def row(r, acc):
                def step(g, acc):
                    base = g * (16 * GUNROLL)
                    for k in range(GUNROLL):
                        col = pl.ds(base + k * 16, 16)
                        x = bx[r, col]
                        y = by[r, col]
                        z = bz[r, col]
                        z2 = x * x + y * y + z * z
                        e = jnp.exp(bvec * z2)
                        acc = acc + jnp.where(z2 < THRESH2, e, 0.0)
                    return acc
                return lax.fori_loop(0, 1024 // (16 * GUNROLL), step, acc)

            acc = lax.fori_loop(0, CROWS, row, acc)
            if c + 1 < NCHUNK:
                descs = nxt

        acc_v[...] = acc
        pltpu.sync_copy(acc_v, part_hbm.at[b])


@jax.jit
def _sc_partials(zt, r):
    kfn = pl.kernel(
        _sc_partials_kernel,
        out_type=jax.ShapeDtypeStruct((BATCH, 16), jnp.float32),
        mesh=plsc.VectorSubcoreMesh(core_axis_name="c", subcore_axis_name="s"),
        scratch_types=[
            pltpu.VMEM((BATCH,), jnp.float32),
            [[pltpu.VMEM((CROWS, 1024), jnp.float32) for _ in range(3)]
             for _ in range(2)],
            pltpu.VMEM((16,), jnp.float32),
            [pltpu.SemaphoreType.DMA, pltpu.SemaphoreType.DMA],
        ],
        compiler_params=pltpu.CompilerParams(
            needs_layout_passes=False, use_tc_tiling_on_sc=True),
    )
    return kfn(zt, r)


def _tc_epilogue_kernel(part_ref, r_ref, n_ref, raw_ref, mu_ref, s2_ref, obj_ref):
    raw = jnp.sum(part_ref[...], axis=1, keepdims=True)
    r = r_ref[...]
    n = n_ref[0, 0]
    a = 1.0 / (r * r)
    lam = (0.5 * THRESH2) * a
    mu_per = (1.0 - jnp.exp(-lam)) / lam
    e2 = (1.0 - jnp.exp(-2.0 * lam)) / (2.0 * lam)
    sig2_per = e2 - mu_per * mu_per
    mu = n * mu_per
    sigma2 = n * sig2_per
    sigma = jnp.sqrt(sigma2)
    raw_ref[...] = raw
    mu_ref[...] = mu
    s2_ref[...] = sigma2
    obj_ref[...] = raw - ALPHA * mu - BETA + sigma


def _tc_epilogue(part, r, num_obs):
    out_shape = jax.ShapeDtypeStruct((BATCH, 1), jnp.float32)
    return pl.pallas_call(
        _tc_epilogue_kernel,
        out_shape=(out_shape, out_shape, out_shape, out_shape),
        in_specs=[
            pl.BlockSpec(memory_space=pltpu.VMEM),
            pl.BlockSpec(memory_space=pltpu.VMEM),
            pl.BlockSpec(memory_space=pltpu.SMEM),
        ],
        out_specs=(
            pl.BlockSpec(memory_space=pltpu.VMEM),
            pl.BlockSpec(memory_space=pltpu.VMEM),
            pl.BlockSpec(memory_space=pltpu.VMEM),
            pl.BlockSpec(memory_space=pltpu.VMEM),
        ),
    )(part, r, num_obs)


def kernel(z, R, num_obs):
    zt = jnp.transpose(z, (0, 3, 1, 2))
    part = _sc_partials(zt, R)
    n2 = jnp.reshape(jnp.asarray(num_obs, jnp.float32), (1, 1))
    raw, mu, sigma2, obj = _tc_epilogue(part, jnp.reshape(R, (BATCH, 1)), n2)
    return (jnp.reshape(raw, (BATCH,)), jnp.reshape(mu, (BATCH,)),
            jnp.reshape(sigma2, (BATCH,)), jnp.reshape(obj, (BATCH,)))


# speedup vs baseline: 262.9702x; 1.2046x over previous
import jax
import jax.numpy as jnp
from jax import lax
from jax.experimental import pallas as pl
from jax.experimental.pallas import tpu as pltpu
from jax.experimental.pallas import tpu_sc as plsc
import numpy as np

BATCH = 64
K_SC = 20
NW = 32

CROWS = 16
NCHUNK = 128 // CROWS
GUNROLL = 8

THRESH = float(2.0 * np.sin(np.deg2rad(2.0) / 2.0))
THRESH2 = THRESH ** 2
ALPHA = 1.0
BETA = 1.0


def _sc_partials_kernel(z_hbm, r_hbm, part_hbm, r_v, bufs, acc_v, sems):
    wid = lax.axis_index("s") * 2 + lax.axis_index("c")
    pltpu.sync_copy(r_hbm, r_v)

    def start(b, c, par):
        return [
            pltpu.async_copy(
                z_hbm.at[b, comp, pl.ds(c * CROWS, CROWS), :],
                bufs[par][comp], sems[par])
            for comp in range(3)
        ]

    n_rows = (K_SC + NW - 1) // NW
    for rr in range(n_rows):
        b = wid + rr * NW
        if rr * NW + NW > K_SC:
            @pl.when(wid < K_SC - rr * NW)
            def _():
                _sc_one_batch(z_hbm, part_hbm, r_v, bufs, acc_v, sems, start, b)
        else:
            _sc_one_batch(z_hbm, part_hbm, r_v, bufs, acc_v, sems, start, b)


def _sc_one_batch(z_hbm, part_hbm, r_v, bufs, acc_v, sems, start, b):
    rvec = plsc.load_gather(r_v, [jnp.full((16,), b, jnp.int32)])
    bvec = -0.5 / (rvec * rvec)

    acc = jnp.zeros((16,), jnp.float32)
    descs = start(b, 0, 0)
    for c in range(NCHUNK):
        par = c % 2
        if c + 1 < NCHUNK:
            nxt = start(b, c + 1, 1 - par)
        for d in descs:
            d.wait()
        bx, by, bz = bufs[par]

        def row(r, acc):
            def step(g, acc):
                base = g * (16 * GUNROLL)
                for k in range(GUNROLL):
                    col = pl.ds(base + k * 16, 16)
                    x = bx[r, col]
                    y = by[r, col]
                    z = bz[r, col]
                    z2 = x * x + y * y + z * z
                    e = jnp.exp(bvec * z2)
                    acc = acc + jnp.where(z2 < THRESH2, e, 0.0)
                return acc
            return lax.fori_loop(0, 1024 // (16 * GUNROLL), step, acc)

        acc = lax.fori_loop(0, CROWS, row, acc)
        if c + 1 < NCHUNK:
            descs = nxt

    acc_v[...] = acc
    pltpu.sync_copy(acc_v, part_hbm.at[b])


@jax.jit
def _sc_partials(zt, r):
    kfn = pl.kernel(
        _sc_partials_kernel,
        out_type=jax.ShapeDtypeStruct((BATCH, 16), jnp.float32),
        mesh=plsc.VectorSubcoreMesh(core_axis_name="c", subcore_axis_name="s"),
        scratch_types=[
            pltpu.VMEM((BATCH,), jnp.float32),
            [[pltpu.VMEM((CROWS, 1024), jnp.float32) for _ in range(3)]
             for _ in range(2)],
            pltpu.VMEM((16,), jnp.float32),
            [pltpu.SemaphoreType.DMA, pltpu.SemaphoreType.DMA],
        ],
        compiler_params=pltpu.CompilerParams(
            needs_layout_passes=False, use_tc_tiling_on_sc=True),
    )
    return kfn(zt, r)


def _tc_reduce_kernel(z_ref, r_ref, out_ref):
    x = z_ref[0, 0]
    y = z_ref[0, 1]
    z = z_ref[0, 2]
    z2 = x * x + y * y + z * z
    rv = r_ref[pl.program_id(0) + K_SC, 0]
    bv = -0.5 / (rv * rv)
    e = jnp.exp(bv * z2)
    s = jnp.sum(jnp.where(z2 < THRESH2, e, 0.0))
    out_ref[pl.program_id(0), 0] = s


def _tc_reduce(zt, r2d):
    n = BATCH - K_SC
    return pl.pallas_call(
        _tc_reduce_kernel,
        grid=(n,),
        in_specs=[
            pl.BlockSpec((1, 3, 128, 1024), lambda i: (i + K_SC, 0, 0, 0)),
            pl.BlockSpec(memory_space=pltpu.SMEM),
        ],
        out_specs=pl.BlockSpec(memory_space=pltpu.SMEM),
        out_shape=jax.ShapeDtypeStruct((n, 1), jnp.float32),
    )(zt, r2d)


def _tc_epilogue_kernel(part_ref, tc_ref, r_ref, n_ref,
                        raw_ref, mu_ref, s2_ref, obj_ref):
    sc_raw = jnp.sum(part_ref[...], axis=1, keepdims=True)
    row = lax.broadcasted_iota(jnp.int32, (BATCH, 1), 0)
    tc_full = jnp.concatenate(
        [jnp.zeros((K_SC, 1), jnp.float32), tc_ref[...]], axis=0)
    raw = jnp.where(row < K_SC, sc_raw, tc_full)
    r = r_ref[...]
    n = n_ref[0, 0]
    a = 1.0 / (r * r)
    lam = (0.5 * THRESH2) * a
    mu_per = (1.0 - jnp.exp(-lam)) / lam
    e2 = (1.0 - jnp.exp(-2.0 * lam)) / (2.0 * lam)
    sig2_per = e2 - mu_per * mu_per
    mu = n * mu_per
    sigma2 = n * sig2_per
    sigma = jnp.sqrt(sigma2)
    raw_ref[...] = raw
    mu_ref[...] = mu
    s2_ref[...] = sigma2
    obj_ref[...] = raw - ALPHA * mu - BETA + sigma


def _tc_epilogue(part, tc_raw, r, num_obs):
    out_shape = jax.ShapeDtypeStruct((BATCH, 1), jnp.float32)
    return pl.pallas_call(
        _tc_epilogue_kernel,
        out_shape=(out_shape, out_shape, out_shape, out_shape),
        in_specs=[
            pl.BlockSpec(memory_space=pltpu.VMEM),
            pl.BlockSpec(memory_space=pltpu.VMEM),
            pl.BlockSpec(memory_space=pltpu.VMEM),
            pl.BlockSpec(memory_space=pltpu.SMEM),
        ],
        out_specs=(
            pl.BlockSpec(memory_space=pltpu.VMEM),
            pl.BlockSpec(memory_space=pltpu.VMEM),
            pl.BlockSpec(memory_space=pltpu.VMEM),
            pl.BlockSpec(memory_space=pltpu.VMEM),
        ),
    )(part, tc_raw, r, num_obs)


def kernel(z, R, num_obs):
    zt = jnp.transpose(z, (0, 3, 1, 2))
    r2d = jnp.reshape(R, (BATCH, 1))
    part = _sc_partials(zt, R)
    tc_raw = _tc_reduce(zt, r2d)
    n2 = jnp.reshape(jnp.asarray(num_obs, jnp.float32), (1, 1))
    raw, mu, sigma2, obj = _tc_epilogue(part, tc_raw, r2d, n2)
    return (jnp.reshape(raw, (BATCH,)), jnp.reshape(mu, (BATCH,)),
            jnp.reshape(sigma2, (BATCH,)), jnp.reshape(obj, (BATCH,)))


# speedup vs baseline: 333.5067x; 1.2682x over previous
import jax
import jax.numpy as jnp
from jax import lax
from jax.experimental import pallas as pl
from jax.experimental.pallas import tpu as pltpu
from jax.experimental.pallas import tpu_sc as plsc
import numpy as np

BATCH = 64
K_SC = 32
NW = 32

CROWS = 16
NCHUNK = 128 // CROWS
GUNROLL = 8

THRESH = float(2.0 * np.sin(np.deg2rad(2.0) / 2.0))
THRESH2 = THRESH ** 2
ALPHA = 1.0
BETA = 1.0


def _sc_partials_kernel(z_hbm, r_hbm, part_hbm, r_v, bufs, acc_v, sems):
    wid = lax.axis_index("s") * 2 + lax.axis_index("c")
    pltpu.sync_copy(r_hbm, r_v)
    b = wid

    def start(c, par):
        return [
            pltpu.async_copy(
                z_hbm.at[b, comp, pl.ds(c * CROWS, CROWS), :],
                bufs[par][comp], sems[par])
            for comp in range(3)
        ]

    rvec = plsc.load_gather(r_v, [jnp.full((16,), b, jnp.int32)])
    bvec = -0.5 / (rvec * rvec)

    acc = jnp.zeros((16,), jnp.float32)
    descs = start(0, 0)
    for c in range(NCHUNK):
        par = c % 2
        if c + 1 < NCHUNK:
            nxt = start(c + 1, 1 - par)
        for d in descs:
            d.wait()
        bx, by, bz = bufs[par]

        def row(r, acc):
            def step(g, acc):
                base = g * (16 * GUNROLL)
                for k in range(GUNROLL):
                    col = pl.ds(base + k * 16, 16)
                    x = bx[r, col]
                    y = by[r, col]
                    z = bz[r, col]
                    z2 = x * x + y * y + z * z
                    e = jnp.exp(bvec * z2)
                    acc = acc + jnp.where(z2 < THRESH2, e, 0.0)
                return acc
            return lax.fori_loop(0, 1024 // (16 * GUNROLL), step, acc)

        acc = lax.fori_loop(0, CROWS, row, acc)
        if c + 1 < NCHUNK:
            descs = nxt

    acc_v[...] = acc
    pltpu.sync_copy(acc_v, part_hbm.at[b])


@jax.jit
def _sc_partials(zt, r):
    kfn = pl.kernel(
        _sc_partials_kernel,
        out_type=jax.ShapeDtypeStruct((K_SC, 16), jnp.float32),
        mesh=plsc.VectorSubcoreMesh(core_axis_name="c", subcore_axis_name="s"),
        scratch_types=[
            pltpu.VMEM((BATCH,), jnp.float32),
            [[pltpu.VMEM((CROWS, 1024), jnp.float32) for _ in range(3)]
             for _ in range(2)],
            pltpu.VMEM((16,), jnp.float32),
            [pltpu.SemaphoreType.DMA, pltpu.SemaphoreType.DMA],
        ],
        compiler_params=pltpu.CompilerParams(
            needs_layout_passes=False, use_tc_tiling_on_sc=True),
    )
    return kfn(zt, r)


def _tc_reduce_kernel(z_ref, r_ref, out_ref):
    x = z_ref[0, 0]
    y = z_ref[0, 1]
    z = z_ref[0, 2]
    z2 = x * x + y * y + z * z
    rv = r_ref[pl.program_id(0) + K_SC]
    bv = -0.5 / (rv * rv)
    e = jnp.exp(bv * z2)
    s = jnp.sum(jnp.where(z2 < THRESH2, e, 0.0))
    out_ref[pl.program_id(0)] = s


def _tc_reduce(zt, r):
    n = BATCH - K_SC
    return pl.pallas_call(
        _tc_reduce_kernel,
        grid=(n,),
        in_specs=[
            pl.BlockSpec((1, 3, 128, 1024), lambda i: (i + K_SC, 0, 0, 0)),
            pl.BlockSpec(memory_space=pltpu.SMEM),
        ],
        out_specs=pl.BlockSpec(memory_space=pltpu.SMEM),
        out_shape=jax.ShapeDtypeStruct((n,), jnp.float32),
    )(zt, r)


def _tc_epilogue_kernel(part_ref, tc_ref, r_ref, n_ref,
                        raw_ref, mu_ref, s2_ref, obj_ref):
    sc_raw = jnp.sum(part_ref[...], axis=1)
    raw = jnp.concatenate([sc_raw, tc_ref[...]], axis=0)
    r = r_ref[...]
    n = n_ref[0]
    a = 1.0 / (r * r)
    lam = (0.5 * THRESH2) * a
    mu_per = (1.0 - jnp.exp(-lam)) / lam
    e2 = (1.0 - jnp.exp(-2.0 * lam)) / (2.0 * lam)
    sig2_per = e2 - mu_per * mu_per
    mu = n * mu_per
    sigma2 = n * sig2_per
    sigma = jnp.sqrt(sigma2)
    raw_ref[...] = raw
    mu_ref[...] = mu
    s2_ref[...] = sigma2
    obj_ref[...] = raw - ALPHA * mu - BETA + sigma


def _tc_epilogue(part, tc_raw, r, num_obs):
    out_shape = jax.ShapeDtypeStruct((BATCH,), jnp.float32)
    return pl.pallas_call(
        _tc_epilogue_kernel,
        out_shape=(out_shape, out_shape, out_shape, out_shape),
        in_specs=[
            pl.BlockSpec(memory_space=pltpu.VMEM),
            pl.BlockSpec(memory_space=pltpu.VMEM),
            pl.BlockSpec(memory_space=pltpu.VMEM),
            pl.BlockSpec(memory_space=pltpu.SMEM),
        ],
        out_specs=(
            pl.BlockSpec(memory_space=pltpu.VMEM),
            pl.BlockSpec(memory_space=pltpu.VMEM),
            pl.BlockSpec(memory_space=pltpu.VMEM),
            pl.BlockSpec(memory_space=pltpu.VMEM),
        ),
    )(part, tc_raw, r, num_obs)


def kernel(z, R, num_obs):
    zt = jnp.transpose(z, (0, 3, 1, 2))
    part = _sc_partials(zt, R)
    tc_raw = _tc_reduce(zt, R)
    n1 = jnp.reshape(jnp.asarray(num_obs, jnp.float32), (1,))
    raw, mu, sigma2, obj = _tc_epilogue(part, tc_raw, R, n1)
    return (raw, mu, sigma2, obj)


# speedup vs baseline: 334.9951x; 1.0045x over previous
import jax
import jax.numpy as jnp
from jax import lax
from jax.experimental import pallas as pl
from jax.experimental.pallas import tpu as pltpu
from jax.experimental.pallas import tpu_sc as plsc
import numpy as np

BATCH = 64
K_SC = 36
NW = 32

CROWS = 16
NCHUNK = 128 // CROWS
GUNROLL = 8

THRESH = float(2.0 * np.sin(np.deg2rad(2.0) / 2.0))
THRESH2 = THRESH ** 2
ALPHA = 1.0
BETA = 1.0


def _sc_partials_kernel(z_hbm, r_hbm, part_hbm, r_v, bufs, acc_v, sems):
    wid = lax.axis_index("s") * 2 + lax.axis_index("c")
    pltpu.sync_copy(r_hbm, r_v)
    b = wid

    def start(c, par):
        return [
            pltpu.async_copy(
                z_hbm.at[b, comp, pl.ds(c * CROWS, CROWS), :],
                bufs[par][comp], sems[par])
            for comp in range(3)
        ]

    rvec = plsc.load_gather(r_v, [jnp.full((16,), b, jnp.int32)])
    bvec = -0.5 / (rvec * rvec)

    acc = jnp.zeros((16,), jnp.float32)
    descs = start(0, 0)
    for c in range(NCHUNK):
        par = c % 2
        if c + 1 < NCHUNK:
            nxt = start(c + 1, 1 - par)
        for d in descs:
            d.wait()
        bx, by, bz = bufs[par]

        def row(r, acc):
            def step(g, acc):
                base = g * (16 * GUNROLL)
                for k in range(GUNROLL):
                    col = pl.ds(base + k * 16, 16)
                    x = bx[r, col]
                    y = by[r, col]
                    z = bz[r, col]
                    z2 = x * x + y * y + z * z
                    e = jnp.exp(bvec * z2)
                    acc = acc + jnp.where(z2 < THRESH2, e, 0.0)
                return acc
            return lax.fori_loop(0, 1024 // (16 * GUNROLL), step, acc)

        acc = lax.fori_loop(0, CROWS, row, acc)
        if c + 1 < NCHUNK:
            descs = nxt

    acc_v[...] = acc
    pltpu.sync_copy(acc_v, part_hbm.at[b])


@jax.jit
def _sc_partials(zt, r):
    kfn = pl.kernel(
        _sc_partials_kernel,
        out_type=jax.ShapeDtypeStruct((K_SC, 16), jnp.float32),
        mesh=plsc.VectorSubcoreMesh(core_axis_name="c", subcore_axis_name="s"),
        scratch_types=[
            pltpu.VMEM((BATCH,), jnp.float32),
            [[pltpu.VMEM((CROWS, 1024), jnp.float32) for _ in range(3)]
             for _ in range(2)],
            pltpu.VMEM((16,), jnp.float32),
            [pltpu.SemaphoreType.DMA, pltpu.SemaphoreType.DMA],
        ],
        compiler_params=pltpu.CompilerParams(
            needs_layout_passes=False, use_tc_tiling_on_sc=True),
    )
    return kfn(zt, r)


def _tc_reduce_kernel(z_ref, r_ref, out_ref):
    x = z_ref[0, 0]
    y = z_ref[0, 1]
    z = z_ref[0, 2]
    z2 = x * x + y * y + z * z
    rv = r_ref[pl.program_id(0) + K_SC]
    bv = -0.5 / (rv * rv)
    e = jnp.exp(bv * z2)
    s = jnp.sum(jnp.where(z2 < THRESH2, e, 0.0))
    out_ref[pl.program_id(0)] = s


def _tc_reduce(zt, r):
    n = BATCH - K_SC
    return pl.pallas_call(
        _tc_reduce_kernel,
        grid=(n,),
        in_specs=[
            pl.BlockSpec((1, 3, 128, 1024), lambda i: (i + K_SC, 0, 0, 0)),
            pl.BlockSpec(memory_space=pltpu.SMEM),
        ],
        out_specs=pl.BlockSpec(memory_space=pltpu.SMEM),
        out_shape=jax.ShapeDtypeStruct((n,), jnp.float32),
    )(zt, r)


def _tc_epilogue_kernel(part_ref, tc_ref, r_ref, n_ref,
                        raw_ref, mu_ref, s2_ref, obj_ref):
    sc_raw = jnp.sum(part_ref[...], axis=1)
    raw = jnp.concatenate([sc_raw, tc_ref[...]], axis=0)
    r = r_ref[...]
    n = n_ref[0]
    a = 1.0 / (r * r)
    lam = (0.5 * THRESH2) * a
    mu_per = (1.0 - jnp.exp(-lam)) / lam
    e2 = (1.0 - jnp.exp(-2.0 * lam)) / (2.0 * lam)
    sig2_per = e2 - mu_per * mu_per
    mu = n * mu_per
    sigma2 = n * sig2_per
    sigma = jnp.sqrt(sigma2)
    raw_ref[...] = raw
    mu_ref[...] = mu
    s2_ref[...] = sigma2
    obj_ref[...] = raw - ALPHA * mu - BETA + sigma


def _tc_epilogue(part, tc_raw, r, num_obs):
    out_shape = jax.ShapeDtypeStruct((BATCH,), jnp.float32)
    return pl.pallas_call(
        _tc_epilogue_kernel,
        out_shape=(out_shape, out_shape, out_shape, out_shape),
        in_specs=[
            pl.BlockSpec(memory_space=pltpu.VMEM),
            pl.BlockSpec(memory_space=pltpu.VMEM),
            pl.BlockSpec(memory_space=pltpu.VMEM),
            pl.BlockSpec(memory_space=pltpu.SMEM),
        ],
        out_specs=(
            pl.BlockSpec(memory_space=pltpu.VMEM),
            pl.BlockSpec(memory_space=pltpu.VMEM),
            pl.BlockSpec(memory_space=pltpu.VMEM),
            pl.BlockSpec(memory_space=pltpu.VMEM),
        ),
    )(part, tc_raw, r, num_obs)


def kernel(z, R, num_obs):
    zt = jnp.transpose(z, (0, 3, 1, 2))
    part = _sc_partials(zt, R)
    tc_raw = _tc_reduce(zt, R)
    n1 = jnp.reshape(jnp.asarray(num_obs, jnp.float32), (1,))
    raw, mu, sigma2, obj = _tc_epilogue(part, tc_raw, R, n1)
    return (raw, mu, sigma2, obj)
